# baseline (speedup 1.0000x reference)
import functools

import jax
import jax.numpy as jnp
from jax import lax
from jax.experimental import pallas as pl
from jax.experimental.pallas import tpu as pltpu
from jax.experimental.pallas import tpu_sc as plsc

NUM_EMBEDDINGS = 100000
OUT_DIM = 128
BATCH = 16384
FIELDS = 26

NC = 2
NS = 16
NW = NC * NS

NB = BATCH // NW
CH = 128
CPF = NB // CH
NBUF = CPF
DIST = 2
NCHUNK = FIELDS * CPF

assert NB * NW == BATCH
assert CPF == NBUF


def _sc_gather(idx_grouped, emb_weight):
    mesh = plsc.VectorSubcoreMesh(
        core_axis_name="c", subcore_axis_name="s", num_cores=NC, num_subcores=NS
    )

    @functools.partial(
        pl.kernel,
        out_type=jax.ShapeDtypeStruct((FIELDS, BATCH, OUT_DIM), jnp.float32),
        mesh=mesh,
        scratch_types=[
            pltpu.VMEM((FIELDS, NB), jnp.int32),
            pltpu.VMEM((NBUF, CH, OUT_DIM), jnp.float32),
        ]
        + [pltpu.SemaphoreType.DMA] * (2 * NBUF),
    )
    def k(idx_hbm, table_hbm, out_hbm, idx_v, rows_v, *sems):
        gsems = sems[:NBUF]
        ssems = sems[NBUF:]
        wid = lax.axis_index("s") * NC + lax.axis_index("c")
        b0 = wid * NB

        pltpu.sync_copy(idx_hbm.at[wid], idx_v)

        def gather(f, cb, kb):
            return pltpu.make_async_copy(
                table_hbm.at[idx_v.at[f, pl.ds(cb * CH, CH)]],
                rows_v.at[kb],
                gsems[kb],
            )

        def store(f, cb, kb):
            return pltpu.make_async_copy(
                rows_v.at[kb],
                out_hbm.at[f, pl.ds(b0 + cb * CH, CH)],
                ssems[kb],
            )

        for kb in range(DIST):
            gather(0, kb, kb).start()

        @pl.loop(0, FIELDS)
        def _(r):
            for k in range(NBUF):
                kp = (k + DIST) % NBUF
                if k < NBUF - DIST:
                    @pl.when(r > 0)
                    def _():
                        store(r - 1, kp, kp).wait()

                    gather(r, kp, kp).start()
                else:
                    store(r, kp, kp).wait()

                    @pl.when(r < FIELDS - 1)
                    def _():
                        gather(r + 1, kp, kp).start()

                gather(r, k, k).wait()
                store(r, k, k).start()

        for kb in range(DIST, NBUF):
            store(FIELDS - 1, kb, kb).wait()

    return k(idx_grouped, emb_weight)


@jax.jit
def kernel(idx, emb_weight):
    idx_grouped = (
        idx.astype(jnp.int32).T.reshape(FIELDS, NW, NB).transpose(1, 0, 2)
    )
    out_fmajor = _sc_gather(idx_grouped, emb_weight)
    return out_fmajor.transpose(1, 0, 2)


# speedup vs baseline: 1.0066x; 1.0066x over previous
import functools

import jax
import jax.numpy as jnp
from jax import lax
from jax.experimental import pallas as pl
from jax.experimental.pallas import tpu as pltpu
from jax.experimental.pallas import tpu_sc as plsc

NUM_EMBEDDINGS = 100000
OUT_DIM = 128
BATCH = 16384
FIELDS = 26

NC = 2
NS = 16
NW = NC * NS

NB = BATCH // NW
CH = 64
CPF = NB // CH
NBUF = 8
DIST = 4
NCHUNK = FIELDS * CPF

assert NB * NW == BATCH
assert CPF == NBUF


def _sc_gather(idx_grouped, emb_weight):
    mesh = plsc.VectorSubcoreMesh(
        core_axis_name="c", subcore_axis_name="s", num_cores=NC, num_subcores=NS
    )

    @functools.partial(
        pl.kernel,
        out_type=jax.ShapeDtypeStruct((FIELDS, BATCH, OUT_DIM), jnp.float32),
        mesh=mesh,
        scratch_types=[
            pltpu.VMEM((FIELDS, NB), jnp.int32),
            pltpu.VMEM((NBUF, CH, OUT_DIM), jnp.float32),
        ]
        + [pltpu.SemaphoreType.DMA] * (2 * NBUF),
    )
    def k(idx_hbm, table_hbm, out_hbm, idx_v, rows_v, *sems):
        gsems = sems[:NBUF]
        ssems = sems[NBUF:]
        wid = lax.axis_index("s") * NC + lax.axis_index("c")
        b0 = wid * NB

        pltpu.sync_copy(idx_hbm.at[wid], idx_v)

        def gather(f, cb, kb):
            return pltpu.make_async_copy(
                table_hbm.at[idx_v.at[f, pl.ds(cb * CH, CH)]],
                rows_v.at[kb],
                gsems[kb],
            )

        def store(f, cb, kb):
            return pltpu.make_async_copy(
                rows_v.at[kb],
                out_hbm.at[f, pl.ds(b0 + cb * CH, CH)],
                ssems[kb],
            )

        for kb in range(DIST):
            gather(0, kb, kb).start()

        @pl.loop(0, FIELDS)
        def _(r):
            for k in range(NBUF):
                kp = (k + DIST) % NBUF
                if k < NBUF - DIST:
                    @pl.when(r > 0)
                    def _():
                        store(r - 1, kp, kp).wait()

                    gather(r, kp, kp).start()
                else:
                    store(r, kp, kp).wait()

                    @pl.when(r < FIELDS - 1)
                    def _():
                        gather(r + 1, kp, kp).start()

                gather(r, k, k).wait()
                store(r, k, k).start()

        for kb in range(DIST, NBUF):
            store(FIELDS - 1, kb, kb).wait()

    return k(idx_grouped, emb_weight)


@jax.jit
def kernel(idx, emb_weight):
    idx_grouped = idx.astype(jnp.int32).reshape(NW, NB, FIELDS).transpose(0, 2, 1)
    out_fmajor = _sc_gather(idx_grouped, emb_weight)
    return out_fmajor.transpose(1, 0, 2)
